# Initial kernel scaffold; baseline (speedup 1.0000x reference)
import jax
import jax.numpy as jnp
from jax.experimental import pallas as pl


def kernel(edge_type, edge_ab, edge_bc, edge_ac, r_index, query_table, fact_table, W, b, ln_scale, ln_bias, W1, b1, W2, b2):
    raise NotImplementedError("write your pallas kernel here")


# speedup vs baseline: 1.1951x; 1.1951x over previous
import functools

import jax
import jax.numpy as jnp
from jax import lax
from jax.experimental import pallas as pl
from jax.experimental.pallas import tpu as pltpu
from jax.experimental.pallas import tpu_sc as plsc

NC = 2
NS = 16
NW = NC * NS
LANES = 16
BLK = 256
C = 64
CB = 128


def _sc_mesh():
    return plsc.VectorSubcoreMesh(
        core_axis_name="c", subcore_axis_name="s", num_cores=NC, num_subcores=NS
    )


def _boundary_kernel(E, D, R2):
    nch = E // CB

    def body(etype_hbm, fact_hbm, out_hbm, it_v, rows_v, sem):
        cid = lax.axis_index("c")
        sid = lax.axis_index("s")
        wid = sid * NC + cid

        def step(k, carry):
            ch = k * NW + wid

            @pl.when(ch < nch)
            def _():
                base = ch * CB
                pltpu.sync_copy(etype_hbm.at[pl.ds(base, CB)], it_v)
                pltpu.async_copy(fact_hbm.at[it_v], rows_v, sem).wait()
                pltpu.sync_copy(rows_v, out_hbm.at[pl.ds(base, CB)])

            return carry

        lax.fori_loop(0, (nch + NW - 1) // NW, step, 0)

    return pl.kernel(
        body,
        out_type=jax.ShapeDtypeStruct((E, D), jnp.float32),
        mesh=_sc_mesh(),
        scratch_types=[
            pltpu.VMEM((CB,), jnp.int32),
            pltpu.VMEM((CB, D), jnp.float32),
            pltpu.SemaphoreType.DMA,
        ],
    )


def _message_kernel(E, D, T_pad, nblk, nbs):
    npass = (nblk + NW - 1) // NW

    def body(hidden_hbm, boundary_hbm, ab_hbm, bc_hbm, acl_hbm, bs_hbm,
             out_hbm, agg_v, ra_v, rb_v, ia_v, ib_v, il_v, bs_v,
             sem_a, sem_b):
        cid = lax.axis_index("c")
        sid = lax.axis_index("s")
        wid = sid * NC + cid
        pltpu.sync_copy(bs_hbm, bs_v)

        def pass_body(p, carry):
            db = p * NW + wid

            @pl.when(db < nblk)
            def _():
                bsv = bs_v[pl.ds(db, LANES)]
                t0 = bsv[0]
                t1 = bsv[1]
                base_row = db * BLK
                pltpu.sync_copy(
                    boundary_hbm.at[pl.ds(base_row, BLK)], agg_v.at[pl.ds(0, BLK)]
                )
                t0a = (t0 // 8) * 8
                nch = (t1 - t0a + C - 1) // C

                def chunk(k, carry2):
                    cb = t0a + k * C
                    pltpu.sync_copy(ab_hbm.at[pl.ds(cb, C)], ia_v)
                    pltpu.sync_copy(bc_hbm.at[pl.ds(cb, C)], ib_v)
                    pltpu.sync_copy(acl_hbm.at[pl.ds(cb, C)], il_v.at[pl.ds(0, C)])
                    cpa = pltpu.async_copy(hidden_hbm.at[ia_v], ra_v, sem_a)
                    cpb = pltpu.async_copy(hidden_hbm.at[ib_v], rb_v, sem_b)
                    cpa.wait()
                    cpb.wait()

                    def tri(t, carry3):
                        tg = cb + t
                        row = il_v[pl.ds(t, LANES)][0]
                        valid = (tg >= t0) & (tg < t1)
                        rowe = jnp.where(valid, row, BLK)
                        for dd in range(D // LANES):
                            s = pl.ds(dd * LANES, LANES)
                            plsc.addupdate(
                                agg_v.at[rowe, s], ra_v[t, s] * rb_v[t, s]
                            )
                        return carry3

                    lax.fori_loop(0, C, tri, 0)
                    return carry2

                lax.fori_loop(0, nch, chunk, 0)
                pltpu.sync_copy(
                    agg_v.at[pl.ds(0, BLK)], out_hbm.at[pl.ds(base_row, BLK)]
                )

            return carry

        lax.fori_loop(0, npass, pass_body, 0)

    return pl.kernel(
        body,
        out_type=jax.ShapeDtypeStruct((E, D), jnp.float32),
        mesh=_sc_mesh(),
        scratch_types=[
            pltpu.VMEM((BLK + 8, D), jnp.float32),
            pltpu.VMEM((C, D), jnp.float32),
            pltpu.VMEM((C, D), jnp.float32),
            pltpu.VMEM((C,), jnp.int32),
            pltpu.VMEM((C,), jnp.int32),
            pltpu.VMEM((C + LANES,), jnp.int32),
            pltpu.VMEM((nbs,), jnp.int32),
            pltpu.SemaphoreType.DMA,
            pltpu.SemaphoreType.DMA,
        ],
        compiler_params=pltpu.CompilerParams(use_tc_tiling_on_sc=False),
    )


def _tc_layer_body(aggb_ref, hid_ref, w_ref, b_ref, s_ref, t_ref, out_ref):
    y = jnp.dot(aggb_ref[...], w_ref[...], preferred_element_type=jnp.float32)
    y = y + b_ref[...]
    mean = jnp.mean(y, axis=1, keepdims=True)
    var = jnp.mean((y - mean) ** 2, axis=1, keepdims=True)
    yn = (y - mean) * lax.rsqrt(var + 1e-5) * s_ref[...] + t_ref[...]
    out_ref[...] = jnp.maximum(yn, 0.0) + hid_ref[...]


def _tc_final_body(hid_ref, w1a_ref, w1b_ref, b1_ref, q_ref, w2_ref, b2_ref,
                   out_ref):
    qv = jnp.dot(q_ref[...], w1b_ref[...], preferred_element_type=jnp.float32)
    qv = qv + b1_ref[...]
    h1 = jnp.dot(hid_ref[...], w1a_ref[...], preferred_element_type=jnp.float32)
    h1 = jnp.maximum(h1 + qv, 0.0)
    out_ref[...] = (
        jnp.dot(h1, w2_ref[...], preferred_element_type=jnp.float32) + b2_ref[...]
    )


def kernel(edge_type, edge_ab, edge_bc, edge_ac, r_index, query_table,
           fact_table, W, b, ln_scale, ln_bias, W1, b1, W2, b2):
    E = edge_type.shape[0]
    T = edge_ab.shape[0]
    D = fact_table.shape[1]
    L = W.shape[0]
    R2 = fact_table.shape[0]

    nblk = E // BLK
    ac32 = edge_ac.astype(jnp.int32)
    order = jnp.argsort(ac32)
    ab_s = edge_ab.astype(jnp.int32)[order]
    bc_s = edge_bc.astype(jnp.int32)[order]
    ac_s = ac32[order]
    acl_s = ac_s - (ac_s // BLK) * BLK
    bs = jnp.searchsorted(ac_s, jnp.arange(nblk + 1, dtype=jnp.int32) * BLK,
                          side="left").astype(jnp.int32)
    nbs = ((nblk + 1 + 15) // 16) * 16
    bs = jnp.concatenate([bs, jnp.full((nbs - (nblk + 1),), T, jnp.int32)])
    T_pad = T + C
    zpad = jnp.zeros((T_pad - T,), jnp.int32)
    ab_s = jnp.concatenate([ab_s, zpad])
    bc_s = jnp.concatenate([bc_s, zpad])
    acl_s = jnp.concatenate([acl_s, zpad])

    boundary = _boundary_kernel(E, D, R2)(edge_type.astype(jnp.int32),
                                          fact_table)

    msg_k = _message_kernel(E, D, T_pad, nblk, nbs)

    RB = 1000
    grid = (E // RB,)
    tc_layer = pl.pallas_call(
        _tc_layer_body,
        grid=grid,
        in_specs=[
            pl.BlockSpec((RB, D), lambda i: (i, 0)),
            pl.BlockSpec((RB, D), lambda i: (i, 0)),
            pl.BlockSpec((D, D), lambda i: (0, 0)),
            pl.BlockSpec((D,), lambda i: (0,)),
            pl.BlockSpec((D,), lambda i: (0,)),
            pl.BlockSpec((D,), lambda i: (0,)),
        ],
        out_specs=pl.BlockSpec((RB, D), lambda i: (i, 0)),
        out_shape=jax.ShapeDtypeStruct((E, D), jnp.float32),
    )

    hidden = boundary
    for i in range(L):
        aggb = msg_k(hidden, boundary, ab_s, bc_s, acl_s, bs)
        hidden = tc_layer(aggb, hidden, W[i], b[i], ln_scale[i], ln_bias[i])

    query = query_table[r_index]
    w1a = W1[:D]
    w1b = W1[D:]
    score = pl.pallas_call(
        _tc_final_body,
        grid=grid,
        in_specs=[
            pl.BlockSpec((RB, D), lambda i: (i, 0)),
            pl.BlockSpec((D, 2 * D), lambda i: (0, 0)),
            pl.BlockSpec((D, 2 * D), lambda i: (0, 0)),
            pl.BlockSpec((2 * D,), lambda i: (0,)),
            pl.BlockSpec((1, D), lambda i: (0, 0)),
            pl.BlockSpec((2 * D, 1), lambda i: (0, 0)),
            pl.BlockSpec((1, 1), lambda i: (0, 0)),
        ],
        out_specs=pl.BlockSpec((RB, 1), lambda i: (i, 0)),
        out_shape=jax.ShapeDtypeStruct((E, 1), jnp.float32),
    )(hidden, w1a, w1b, b1, query, W2, b2.reshape(1, 1))
    return score


# speedup vs baseline: 1.7165x; 1.4363x over previous
import functools

import jax
import jax.numpy as jnp
from jax import lax
from jax.experimental import pallas as pl
from jax.experimental.pallas import tpu as pltpu
from jax.experimental.pallas import tpu_sc as plsc

NC = 2
NS = 16
NW = NC * NS
LANES = 16
BLK = 250
C = 48
CB = 128


def _sc_mesh():
    return plsc.VectorSubcoreMesh(
        core_axis_name="c", subcore_axis_name="s", num_cores=NC, num_subcores=NS
    )


def _boundary_kernel(E, D, R2):
    nch = E // CB

    def body(etype_hbm, fact_hbm, out_hbm, it_v, rows_v, sem):
        cid = lax.axis_index("c")
        sid = lax.axis_index("s")
        wid = sid * NC + cid

        def step(k, carry):
            ch = k * NW + wid

            @pl.when(ch < nch)
            def _():
                base = ch * CB
                pltpu.sync_copy(etype_hbm.at[pl.ds(base, CB)], it_v)
                pltpu.async_copy(fact_hbm.at[it_v], rows_v, sem).wait()
                pltpu.sync_copy(rows_v, out_hbm.at[pl.ds(base, CB)])

            return carry

        lax.fori_loop(0, (nch + NW - 1) // NW, step, 0)

    return pl.kernel(
        body,
        out_type=jax.ShapeDtypeStruct((E, D), jnp.float32),
        mesh=_sc_mesh(),
        scratch_types=[
            pltpu.VMEM((CB,), jnp.int32),
            pltpu.VMEM((CB, D), jnp.float32),
            pltpu.SemaphoreType.DMA,
        ],
    )


def _message_kernel(E, D, T_pad, nblk, nbs):
    npass = (nblk + NW - 1) // NW

    STG = 50
    TRASH = NS * BLK

    def body(hidden_hbm, boundary_hbm, ab_hbm, bc_hbm, acl_hbm, bs_hbm,
             out_hbm, agg_sh, ra0_v, rb0_v, ra1_v, rb1_v, stg_v,
             ia0_v, ib0_v, il0_v, ia1_v, ib1_v, il1_v, bs_v,
             sem_a0, sem_b0, sem_a1, sem_b1):
        cid = lax.axis_index("c")
        sid = lax.axis_index("s")
        wid = sid * NC + cid
        pltpu.sync_copy(bs_hbm, bs_v)
        iota = lax.broadcasted_iota(jnp.int32, (LANES,), 0)
        ras = (ra0_v, ra1_v)
        rbs = (rb0_v, rb1_v)
        ias = (ia0_v, ia1_v)
        ibs = (ib0_v, ib1_v)
        ils = (il0_v, il1_v)
        sas = (sem_a0, sem_a1)
        sbs = (sem_b0, sem_b1)

        def pass_body(p, carry):
            db = p * NW + wid

            @pl.when(db < nblk)
            def _():
                bsv = bs_v[pl.ds(db, LANES)]
                t0 = bsv[0]
                t1 = bsv[1]
                base_row = db * BLK
                arow = sid * BLK
                for j in range(BLK // STG):
                    pltpu.sync_copy(
                        boundary_hbm.at[pl.ds(base_row + j * STG, STG)], stg_v
                    )
                    pltpu.sync_copy(stg_v, agg_sh.at[pl.ds(arow + j * STG, STG)])
                t0a = (t0 // 8) * 8
                nch = (t1 - t0a + C - 1) // C
                t0v = jnp.broadcast_to(t0, (LANES,))
                t1v = jnp.broadcast_to(t1, (LANES,))

                def fire(k, bb):
                    @pl.when(k < nch)
                    def _():
                        cb = t0a + k * C
                        pltpu.sync_copy(ab_hbm.at[pl.ds(cb, C)], ias[bb])
                        pltpu.sync_copy(bc_hbm.at[pl.ds(cb, C)], ibs[bb])
                        pltpu.sync_copy(acl_hbm.at[pl.ds(cb, C)], ils[bb])
                        pltpu.async_copy(hidden_hbm.at[ias[bb]], ras[bb], sas[bb])
                        pltpu.async_copy(hidden_hbm.at[ibs[bb]], rbs[bb], sbs[bb])
                        for q in range(C // LANES):
                            s = pl.ds(q * LANES, LANES)
                            tgv = iota + (cb + q * LANES)
                            ok = (tgv >= t0v) & (tgv < t1v)
                            ils[bb][s] = jnp.where(
                                ok, ils[bb][s] + arow, TRASH
                            )

                def consume(k, bb):
                    @pl.when(k < nch)
                    def _():
                        pltpu.make_async_copy(
                            hidden_hbm.at[ias[bb]], ras[bb], sas[bb]
                        ).wait()
                        pltpu.make_async_copy(
                            hidden_hbm.at[ibs[bb]], rbs[bb], sbs[bb]
                        ).wait()

                        def mrow(r, carry3):
                            for dd in range(D // LANES):
                                s = pl.ds(dd * LANES, LANES)
                                ras[bb][r, s] = ras[bb][r, s] * rbs[bb][r, s]
                            return carry3

                        lax.fori_loop(0, C, mrow, 0)
                        pltpu.sync_copy(ras[bb], agg_sh.at[ils[bb]], add=True)

                fire(0, 0)
                fire(1, 1)

                def pair(g, carry2):
                    k = 2 * g
                    consume(k, 0)
                    fire(k + 2, 0)
                    consume(k + 1, 1)
                    fire(k + 3, 1)
                    return carry2

                lax.fori_loop(0, (nch + 1) // 2, pair, 0)
                for j in range(BLK // STG):
                    pltpu.sync_copy(agg_sh.at[pl.ds(arow + j * STG, STG)], stg_v)
                    pltpu.sync_copy(
                        stg_v, out_hbm.at[pl.ds(base_row + j * STG, STG)]
                    )

            return carry

        lax.fori_loop(0, npass, pass_body, 0)

    return pl.kernel(
        body,
        out_type=jax.ShapeDtypeStruct((E, D), jnp.float32),
        mesh=_sc_mesh(),
        scratch_types=[
            pltpu.VMEM_SHARED((NS * BLK + 8, D), jnp.float32),
            pltpu.VMEM((C, D), jnp.float32),
            pltpu.VMEM((C, D), jnp.float32),
            pltpu.VMEM((C, D), jnp.float32),
            pltpu.VMEM((C, D), jnp.float32),
            pltpu.VMEM((STG, D), jnp.float32),
            pltpu.VMEM((C,), jnp.int32),
            pltpu.VMEM((C,), jnp.int32),
            pltpu.VMEM((C,), jnp.int32),
            pltpu.VMEM((C,), jnp.int32),
            pltpu.VMEM((C,), jnp.int32),
            pltpu.VMEM((C,), jnp.int32),
            pltpu.VMEM((nbs,), jnp.int32),
            pltpu.SemaphoreType.DMA,
            pltpu.SemaphoreType.DMA,
            pltpu.SemaphoreType.DMA,
            pltpu.SemaphoreType.DMA,
        ],
        compiler_params=pltpu.CompilerParams(use_tc_tiling_on_sc=False),
    )


def _tc_layer_body(aggb_ref, hid_ref, w_ref, b_ref, s_ref, t_ref, out_ref):
    y = jnp.dot(aggb_ref[...], w_ref[...], preferred_element_type=jnp.float32)
    y = y + b_ref[...]
    mean = jnp.mean(y, axis=1, keepdims=True)
    var = jnp.mean((y - mean) ** 2, axis=1, keepdims=True)
    yn = (y - mean) * lax.rsqrt(var + 1e-5) * s_ref[...] + t_ref[...]
    out_ref[...] = jnp.maximum(yn, 0.0) + hid_ref[...]


def _tc_final_body(hid_ref, w1a_ref, w1b_ref, b1_ref, q_ref, w2_ref, b2_ref,
                   out_ref):
    qv = jnp.dot(q_ref[...], w1b_ref[...], preferred_element_type=jnp.float32)
    qv = qv + b1_ref[...]
    h1 = jnp.dot(hid_ref[...], w1a_ref[...], preferred_element_type=jnp.float32)
    h1 = jnp.maximum(h1 + qv, 0.0)
    out_ref[...] = (
        jnp.dot(h1, w2_ref[...], preferred_element_type=jnp.float32) + b2_ref[...]
    )


def kernel(edge_type, edge_ab, edge_bc, edge_ac, r_index, query_table,
           fact_table, W, b, ln_scale, ln_bias, W1, b1, W2, b2):
    E = edge_type.shape[0]
    T = edge_ab.shape[0]
    D = fact_table.shape[1]
    L = W.shape[0]
    R2 = fact_table.shape[0]

    nblk = E // BLK
    ac32 = edge_ac.astype(jnp.int32)
    order = jnp.argsort(ac32)
    ab_s = edge_ab.astype(jnp.int32)[order]
    bc_s = edge_bc.astype(jnp.int32)[order]
    ac_s = ac32[order]
    acl_s = ac_s - (ac_s // BLK) * BLK
    bs = jnp.searchsorted(ac_s, jnp.arange(nblk + 1, dtype=jnp.int32) * BLK,
                          side="left").astype(jnp.int32)
    nbs = ((nblk + 1 + 15) // 16) * 16
    bs = jnp.concatenate([bs, jnp.full((nbs - (nblk + 1),), T, jnp.int32)])
    T_pad = T + C
    zpad = jnp.zeros((T_pad - T,), jnp.int32)
    ab_s = jnp.concatenate([ab_s, zpad])
    bc_s = jnp.concatenate([bc_s, zpad])
    acl_s = jnp.concatenate([acl_s, zpad])

    boundary = _boundary_kernel(E, D, R2)(edge_type.astype(jnp.int32),
                                          fact_table)

    msg_k = _message_kernel(E, D, T_pad, nblk, nbs)

    RB = 1000
    grid = (E // RB,)
    tc_layer = pl.pallas_call(
        _tc_layer_body,
        grid=grid,
        in_specs=[
            pl.BlockSpec((RB, D), lambda i: (i, 0)),
            pl.BlockSpec((RB, D), lambda i: (i, 0)),
            pl.BlockSpec((D, D), lambda i: (0, 0)),
            pl.BlockSpec((D,), lambda i: (0,)),
            pl.BlockSpec((D,), lambda i: (0,)),
            pl.BlockSpec((D,), lambda i: (0,)),
        ],
        out_specs=pl.BlockSpec((RB, D), lambda i: (i, 0)),
        out_shape=jax.ShapeDtypeStruct((E, D), jnp.float32),
    )

    hidden = boundary
    for i in range(L):
        aggb = msg_k(hidden, boundary, ab_s, bc_s, acl_s, bs)
        hidden = tc_layer(aggb, hidden, W[i], b[i], ln_scale[i], ln_bias[i])

    query = query_table[r_index]
    w1a = W1[:D]
    w1b = W1[D:]
    score = pl.pallas_call(
        _tc_final_body,
        grid=grid,
        in_specs=[
            pl.BlockSpec((RB, D), lambda i: (i, 0)),
            pl.BlockSpec((D, 2 * D), lambda i: (0, 0)),
            pl.BlockSpec((D, 2 * D), lambda i: (0, 0)),
            pl.BlockSpec((2 * D,), lambda i: (0,)),
            pl.BlockSpec((1, D), lambda i: (0, 0)),
            pl.BlockSpec((2 * D, 1), lambda i: (0, 0)),
            pl.BlockSpec((1, 1), lambda i: (0, 0)),
        ],
        out_specs=pl.BlockSpec((RB, 1), lambda i: (i, 0)),
        out_shape=jax.ShapeDtypeStruct((E, 1), jnp.float32),
    )(hidden, w1a, w1b, b1, query, W2, b2.reshape(1, 1))
    return score


# speedup vs baseline: 2.0089x; 1.1703x over previous
import functools

import jax
import jax.numpy as jnp
from jax import lax
from jax.experimental import pallas as pl
from jax.experimental.pallas import tpu as pltpu
from jax.experimental.pallas import tpu_sc as plsc

NC = 2
NS = 16
NW = NC * NS
LANES = 16
BLK = 250
C = 48
CB = 128


def _sc_mesh():
    return plsc.VectorSubcoreMesh(
        core_axis_name="c", subcore_axis_name="s", num_cores=NC, num_subcores=NS
    )


def _tc_boundary_body(type_ref, factp_ref, out_ref):
    rb = type_ref.shape[0]
    kp = factp_ref.shape[0]
    oh = (type_ref[...] == lax.broadcasted_iota(jnp.int32, (rb, kp), 1))
    out_ref[...] = jnp.dot(oh.astype(jnp.float32), factp_ref[...],
                           preferred_element_type=jnp.float32)


def _message_kernel(E, D, T_pad, nblk, nbs):
    npass = (nblk + NW - 1) // NW

    STG = 50
    TRASH = NS * BLK

    def body(hidden_hbm, boundary_hbm, idx3_hbm, bs_hbm,
             out_hbm, agg_sh, ra0_v, rb0_v, ra1_v, rb1_v, stg_v,
             ix0_v, ix1_v, il0_v, il1_v, bs_v,
             sem_i0, sem_i1, sem_a0, sem_b0, sem_a1, sem_b1):
        cid = lax.axis_index("c")
        sid = lax.axis_index("s")
        wid = sid * NC + cid
        pltpu.sync_copy(bs_hbm, bs_v)
        iota = lax.broadcasted_iota(jnp.int32, (LANES,), 0)
        ras = (ra0_v, ra1_v)
        rbs = (rb0_v, rb1_v)
        ixs = (ix0_v, ix1_v)
        ils = (il0_v, il1_v)
        sis = (sem_i0, sem_i1)
        sas = (sem_a0, sem_a1)
        sbs = (sem_b0, sem_b1)

        def pass_body(p, carry):
            db = p * NW + wid

            @pl.when(db < nblk)
            def _():
                bsv = bs_v[pl.ds(db, LANES)]
                t0 = bsv[0]
                t1 = bsv[1]
                base_row = db * BLK
                arow = sid * BLK
                for j in range(BLK // STG):
                    pltpu.sync_copy(
                        boundary_hbm.at[pl.ds(base_row + j * STG, STG)], stg_v
                    )
                    pltpu.sync_copy(stg_v, agg_sh.at[pl.ds(arow + j * STG, STG)])
                t0a = (t0 // 8) * 8
                nch = (t1 - t0a + C - 1) // C
                t0v = jnp.broadcast_to(t0, (LANES,))
                t1v = jnp.broadcast_to(t1, (LANES,))

                def fire_idx(k, bb):
                    @pl.when(k < nch)
                    def _():
                        cb = t0a + k * C
                        pltpu.async_copy(
                            idx3_hbm.at[:, pl.ds(cb, C)], ixs[bb], sis[bb]
                        )

                def fire_rows(k, bb):
                    @pl.when(k < nch)
                    def _():
                        cb = t0a + k * C
                        pltpu.make_async_copy(
                            idx3_hbm.at[:, pl.ds(cb, C)], ixs[bb], sis[bb]
                        ).wait()
                        pltpu.async_copy(
                            hidden_hbm.at[ixs[bb].at[0]], ras[bb], sas[bb]
                        )
                        pltpu.async_copy(
                            hidden_hbm.at[ixs[bb].at[1]], rbs[bb], sbs[bb]
                        )
                        for q in range(C // LANES):
                            s = pl.ds(q * LANES, LANES)
                            tgv = iota + (cb + q * LANES)
                            ok = (tgv >= t0v) & (tgv < t1v)
                            ils[bb][s] = jnp.where(
                                ok, ixs[bb][2, s] + arow, TRASH
                            )

                def wait_rows(k, bb):
                    @pl.when(k < nch)
                    def _():
                        pltpu.make_async_copy(
                            hidden_hbm.at[ixs[bb].at[0]], ras[bb], sas[bb]
                        ).wait()
                        pltpu.make_async_copy(
                            hidden_hbm.at[ixs[bb].at[1]], rbs[bb], sbs[bb]
                        ).wait()

                def compute(k, bb):
                    @pl.when(k < nch)
                    def _():
                        def mrow(r4, carry3):
                            for r16 in range(4):
                                r = r4 * 4 + r16
                                for dd in range(D // LANES):
                                    s = pl.ds(dd * LANES, LANES)
                                    ras[bb][r, s] = ras[bb][r, s] * rbs[bb][r, s]
                            return carry3

                        lax.fori_loop(0, C // 4, mrow, 0)
                        pltpu.sync_copy(ras[bb], agg_sh.at[ils[bb]], add=True)

                fire_idx(0, 0)
                fire_idx(1, 1)
                fire_rows(0, 0)

                def pair(g, carry2):
                    k = 2 * g
                    wait_rows(k, 0)
                    fire_idx(k + 2, 0)
                    fire_rows(k + 1, 1)
                    compute(k, 0)
                    wait_rows(k + 1, 1)
                    fire_idx(k + 3, 1)
                    fire_rows(k + 2, 0)
                    compute(k + 1, 1)
                    return carry2

                lax.fori_loop(0, (nch + 1) // 2, pair, 0)
                for j in range(BLK // STG):
                    pltpu.sync_copy(agg_sh.at[pl.ds(arow + j * STG, STG)], stg_v)
                    pltpu.sync_copy(
                        stg_v, out_hbm.at[pl.ds(base_row + j * STG, STG)]
                    )

            return carry

        lax.fori_loop(0, npass, pass_body, 0)

    return pl.kernel(
        body,
        out_type=jax.ShapeDtypeStruct((E, D), jnp.float32),
        mesh=_sc_mesh(),
        scratch_types=[
            pltpu.VMEM_SHARED((NS * BLK + 8, D), jnp.float32),
            pltpu.VMEM((C, D), jnp.float32),
            pltpu.VMEM((C, D), jnp.float32),
            pltpu.VMEM((C, D), jnp.float32),
            pltpu.VMEM((C, D), jnp.float32),
            pltpu.VMEM((STG, D), jnp.float32),
            pltpu.VMEM((3, C), jnp.int32),
            pltpu.VMEM((3, C), jnp.int32),
            pltpu.VMEM((C,), jnp.int32),
            pltpu.VMEM((C,), jnp.int32),
            pltpu.VMEM((nbs,), jnp.int32),
            pltpu.SemaphoreType.DMA,
            pltpu.SemaphoreType.DMA,
            pltpu.SemaphoreType.DMA,
            pltpu.SemaphoreType.DMA,
            pltpu.SemaphoreType.DMA,
            pltpu.SemaphoreType.DMA,
        ],
        compiler_params=pltpu.CompilerParams(use_tc_tiling_on_sc=False),
    )


def _tc_layer_body(aggb_ref, hid_ref, w_ref, b_ref, s_ref, t_ref, out_ref):
    y = jnp.dot(aggb_ref[...], w_ref[...], preferred_element_type=jnp.float32)
    y = y + b_ref[...]
    mean = jnp.mean(y, axis=1, keepdims=True)
    var = jnp.mean((y - mean) ** 2, axis=1, keepdims=True)
    yn = (y - mean) * lax.rsqrt(var + 1e-5) * s_ref[...] + t_ref[...]
    out_ref[...] = jnp.maximum(yn, 0.0) + hid_ref[...]


def _tc_final_body(hid_ref, w1a_ref, w1b_ref, b1_ref, q_ref, w2_ref, b2_ref,
                   out_ref):
    qv = jnp.dot(q_ref[...], w1b_ref[...], preferred_element_type=jnp.float32)
    qv = qv + b1_ref[...]
    h1 = jnp.dot(hid_ref[...], w1a_ref[...], preferred_element_type=jnp.float32)
    h1 = jnp.maximum(h1 + qv, 0.0)
    out_ref[...] = (
        jnp.dot(h1, w2_ref[...], preferred_element_type=jnp.float32) + b2_ref[...]
    )


def kernel(edge_type, edge_ab, edge_bc, edge_ac, r_index, query_table,
           fact_table, W, b, ln_scale, ln_bias, W1, b1, W2, b2):
    E = edge_type.shape[0]
    T = edge_ab.shape[0]
    D = fact_table.shape[1]
    L = W.shape[0]
    R2 = fact_table.shape[0]

    nblk = E // BLK
    ac32 = edge_ac.astype(jnp.int32)
    order = jnp.argsort(ac32)
    ab_s = edge_ab.astype(jnp.int32)[order]
    bc_s = edge_bc.astype(jnp.int32)[order]
    ac_s = ac32[order]
    acl_s = ac_s - (ac_s // BLK) * BLK
    bs = jnp.searchsorted(ac_s, jnp.arange(nblk + 1, dtype=jnp.int32) * BLK,
                          side="left").astype(jnp.int32)
    nbs = ((nblk + 1 + 15) // 16) * 16
    bs = jnp.concatenate([bs, jnp.full((nbs - (nblk + 1),), T, jnp.int32)])
    T_pad = T + C
    zpad = jnp.zeros((T_pad - T,), jnp.int32)
    ab_s = jnp.concatenate([ab_s, zpad])
    bc_s = jnp.concatenate([bc_s, zpad])
    acl_s = jnp.concatenate([acl_s, zpad])
    idx3 = jnp.stack([ab_s, bc_s, acl_s])

    msg_k = _message_kernel(E, D, T_pad, nblk, nbs)

    RB = 1000
    grid = (E // RB,)
    tc_layer = pl.pallas_call(
        _tc_layer_body,
        grid=grid,
        in_specs=[
            pl.BlockSpec((RB, D), lambda i: (i, 0)),
            pl.BlockSpec((RB, D), lambda i: (i, 0)),
            pl.BlockSpec((D, D), lambda i: (0, 0)),
            pl.BlockSpec((D,), lambda i: (0,)),
            pl.BlockSpec((D,), lambda i: (0,)),
            pl.BlockSpec((D,), lambda i: (0,)),
        ],
        out_specs=pl.BlockSpec((RB, D), lambda i: (i, 0)),
        out_shape=jax.ShapeDtypeStruct((E, D), jnp.float32),
    )

    KP = 128
    factp = jnp.zeros((KP, D), jnp.float32).at[:R2].set(fact_table)
    boundary = pl.pallas_call(
        _tc_boundary_body,
        grid=grid,
        in_specs=[
            pl.BlockSpec((RB, 1), lambda i: (i, 0)),
            pl.BlockSpec((KP, D), lambda i: (0, 0)),
        ],
        out_specs=pl.BlockSpec((RB, D), lambda i: (i, 0)),
        out_shape=jax.ShapeDtypeStruct((E, D), jnp.float32),
    )(edge_type.astype(jnp.int32).reshape(E, 1), factp)

    hidden = boundary
    for i in range(L):
        aggb = msg_k(hidden, boundary, idx3, bs)
        hidden = tc_layer(aggb, hidden, W[i], b[i], ln_scale[i], ln_bias[i])

    query = query_table[r_index]
    w1a = W1[:D]
    w1b = W1[D:]
    score = pl.pallas_call(
        _tc_final_body,
        grid=grid,
        in_specs=[
            pl.BlockSpec((RB, D), lambda i: (i, 0)),
            pl.BlockSpec((D, 2 * D), lambda i: (0, 0)),
            pl.BlockSpec((D, 2 * D), lambda i: (0, 0)),
            pl.BlockSpec((2 * D,), lambda i: (0,)),
            pl.BlockSpec((1, D), lambda i: (0, 0)),
            pl.BlockSpec((2 * D, 1), lambda i: (0, 0)),
            pl.BlockSpec((1, 1), lambda i: (0, 0)),
        ],
        out_specs=pl.BlockSpec((RB, 1), lambda i: (i, 0)),
        out_shape=jax.ShapeDtypeStruct((E, 1), jnp.float32),
    )(hidden, w1a, w1b, b1, query, W2, b2.reshape(1, 1))
    return score


# speedup vs baseline: 2.0478x; 1.0193x over previous
import functools

import jax
import jax.numpy as jnp
from jax import lax
from jax.experimental import pallas as pl
from jax.experimental.pallas import tpu as pltpu
from jax.experimental.pallas import tpu_sc as plsc

NC = 2
NS = 16
NW = NC * NS
LANES = 16
BLK = 250
C = 48
CB = 128


def _sc_mesh():
    return plsc.VectorSubcoreMesh(
        core_axis_name="c", subcore_axis_name="s", num_cores=NC, num_subcores=NS
    )


def _tc_boundary_body(type_ref, factp_ref, out_ref):
    rb = type_ref.shape[0]
    kp = factp_ref.shape[0]
    oh = (type_ref[...] == lax.broadcasted_iota(jnp.int32, (rb, kp), 1))
    out_ref[...] = jnp.dot(oh.astype(jnp.float32), factp_ref[...],
                           preferred_element_type=jnp.float32)


def _message_kernel(E, D, T_pad, nblk, nbs):
    npass = (nblk + NW - 1) // NW

    STG = 50
    TRASH = NS * BLK

    def body(hidden_hbm, boundary_hbm, idx3_hbm, bs_hbm,
             out_hbm, agg_sh, ra0_v, rb0_v, ra1_v, rb1_v, stg_v,
             ix0_v, ix1_v, il0_v, il1_v, bs_v,
             sem_i0, sem_i1, sem_a0, sem_b0, sem_a1, sem_b1,
             sem_s0, sem_s1):
        cid = lax.axis_index("c")
        sid = lax.axis_index("s")
        wid = sid * NC + cid
        pltpu.sync_copy(bs_hbm, bs_v)
        iota = lax.broadcasted_iota(jnp.int32, (LANES,), 0)
        ras = (ra0_v, ra1_v)
        rbs = (rb0_v, rb1_v)
        ixs = (ix0_v, ix1_v)
        ils = (il0_v, il1_v)
        sis = (sem_i0, sem_i1)
        sas = (sem_a0, sem_a1)
        sbs = (sem_b0, sem_b1)
        sss = (sem_s0, sem_s1)

        def pass_body(p, carry):
            db = p * NW + wid

            @pl.when(db < nblk)
            def _():
                bsv = bs_v[pl.ds(db, LANES)]
                t0 = bsv[0]
                t1 = bsv[1]
                base_row = db * BLK
                arow = sid * BLK
                for j in range(BLK // STG):
                    pltpu.sync_copy(
                        boundary_hbm.at[pl.ds(base_row + j * STG, STG)], stg_v
                    )
                    pltpu.sync_copy(stg_v, agg_sh.at[pl.ds(arow + j * STG, STG)])
                t0a = (t0 // 8) * 8
                nch = (t1 - t0a + C - 1) // C
                t0v = jnp.broadcast_to(t0, (LANES,))
                t1v = jnp.broadcast_to(t1, (LANES,))

                def fire_idx(k, bb):
                    @pl.when(k < nch)
                    def _():
                        cb = t0a + k * C
                        pltpu.async_copy(
                            idx3_hbm.at[:, pl.ds(cb, C)], ixs[bb], sis[bb]
                        )

                def fire_rows(k, bb):
                    @pl.when(k < nch)
                    def _():
                        cb = t0a + k * C

                        @pl.when(k >= 2)
                        def _():
                            pltpu.make_async_copy(
                                ras[bb], agg_sh.at[ils[bb]], sss[bb]
                            ).wait()

                        pltpu.make_async_copy(
                            idx3_hbm.at[:, pl.ds(cb, C)], ixs[bb], sis[bb]
                        ).wait()
                        pltpu.async_copy(
                            hidden_hbm.at[ixs[bb].at[0]], ras[bb], sas[bb]
                        )
                        pltpu.async_copy(
                            hidden_hbm.at[ixs[bb].at[1]], rbs[bb], sbs[bb]
                        )
                        for q in range(C // LANES):
                            s = pl.ds(q * LANES, LANES)
                            tgv = iota + (cb + q * LANES)
                            ok = (tgv >= t0v) & (tgv < t1v)
                            ils[bb][s] = jnp.where(
                                ok, ixs[bb][2, s] + arow, TRASH
                            )

                def wait_rows(k, bb):
                    @pl.when(k < nch)
                    def _():
                        pltpu.make_async_copy(
                            hidden_hbm.at[ixs[bb].at[0]], ras[bb], sas[bb]
                        ).wait()
                        pltpu.make_async_copy(
                            hidden_hbm.at[ixs[bb].at[1]], rbs[bb], sbs[bb]
                        ).wait()

                def compute(k, bb):
                    @pl.when(k < nch)
                    def _():
                        def mrow(r4, carry3):
                            for r16 in range(4):
                                r = r4 * 4 + r16
                                for dd in range(D // LANES):
                                    s = pl.ds(dd * LANES, LANES)
                                    ras[bb][r, s] = ras[bb][r, s] * rbs[bb][r, s]
                            return carry3

                        lax.fori_loop(0, C // 4, mrow, 0)
                        pltpu.async_copy(
                            ras[bb], agg_sh.at[ils[bb]], sss[bb], add=True
                        )

                fire_idx(0, 0)
                fire_idx(1, 1)
                fire_rows(0, 0)

                def pair(g, carry2):
                    k = 2 * g
                    wait_rows(k, 0)
                    fire_idx(k + 2, 0)
                    fire_rows(k + 1, 1)
                    compute(k, 0)
                    wait_rows(k + 1, 1)
                    fire_idx(k + 3, 1)
                    fire_rows(k + 2, 0)
                    compute(k + 1, 1)
                    return carry2

                lax.fori_loop(0, (nch + 1) // 2, pair, 0)

                @pl.when(nch >= 1)
                def _():
                    pltpu.make_async_copy(
                        ras[0], agg_sh.at[ils[0]], sss[0]
                    ).wait()

                @pl.when(nch >= 2)
                def _():
                    pltpu.make_async_copy(
                        ras[1], agg_sh.at[ils[1]], sss[1]
                    ).wait()

                for j in range(BLK // STG):
                    pltpu.sync_copy(agg_sh.at[pl.ds(arow + j * STG, STG)], stg_v)
                    pltpu.sync_copy(
                        stg_v, out_hbm.at[pl.ds(base_row + j * STG, STG)]
                    )

            return carry

        lax.fori_loop(0, npass, pass_body, 0)

    return pl.kernel(
        body,
        out_type=jax.ShapeDtypeStruct((E, D), jnp.float32),
        mesh=_sc_mesh(),
        scratch_types=[
            pltpu.VMEM_SHARED((NS * BLK + 8, D), jnp.float32),
            pltpu.VMEM((C, D), jnp.float32),
            pltpu.VMEM((C, D), jnp.float32),
            pltpu.VMEM((C, D), jnp.float32),
            pltpu.VMEM((C, D), jnp.float32),
            pltpu.VMEM((STG, D), jnp.float32),
            pltpu.VMEM((3, C), jnp.int32),
            pltpu.VMEM((3, C), jnp.int32),
            pltpu.VMEM((C,), jnp.int32),
            pltpu.VMEM((C,), jnp.int32),
            pltpu.VMEM((nbs,), jnp.int32),
            pltpu.SemaphoreType.DMA,
            pltpu.SemaphoreType.DMA,
            pltpu.SemaphoreType.DMA,
            pltpu.SemaphoreType.DMA,
            pltpu.SemaphoreType.DMA,
            pltpu.SemaphoreType.DMA,
            pltpu.SemaphoreType.DMA,
            pltpu.SemaphoreType.DMA,
        ],
        compiler_params=pltpu.CompilerParams(use_tc_tiling_on_sc=False),
    )


def _tc_layer_body(aggb_ref, hid_ref, w_ref, b_ref, s_ref, t_ref, out_ref):
    y = jnp.dot(aggb_ref[...], w_ref[...], preferred_element_type=jnp.float32)
    y = y + b_ref[...]
    mean = jnp.mean(y, axis=1, keepdims=True)
    var = jnp.mean((y - mean) ** 2, axis=1, keepdims=True)
    yn = (y - mean) * lax.rsqrt(var + 1e-5) * s_ref[...] + t_ref[...]
    out_ref[...] = jnp.maximum(yn, 0.0) + hid_ref[...]


def _tc_final_body(aggb_ref, hid_ref, w_ref, b_ref, s_ref, t_ref,
                   w1a_ref, w1b_ref, b1_ref, q_ref, w2_ref, b2_ref, out_ref):
    y = jnp.dot(aggb_ref[...], w_ref[...], preferred_element_type=jnp.float32)
    y = y + b_ref[...]
    mean = jnp.mean(y, axis=1, keepdims=True)
    var = jnp.mean((y - mean) ** 2, axis=1, keepdims=True)
    yn = (y - mean) * lax.rsqrt(var + 1e-5) * s_ref[...] + t_ref[...]
    hid = jnp.maximum(yn, 0.0) + hid_ref[...]
    qv = jnp.dot(q_ref[...], w1b_ref[...], preferred_element_type=jnp.float32)
    qv = qv + b1_ref[...]
    h1 = jnp.dot(hid, w1a_ref[...], preferred_element_type=jnp.float32)
    h1 = jnp.maximum(h1 + qv, 0.0)
    out_ref[...] = (
        jnp.dot(h1, w2_ref[...], preferred_element_type=jnp.float32) + b2_ref[...]
    )


def kernel(edge_type, edge_ab, edge_bc, edge_ac, r_index, query_table,
           fact_table, W, b, ln_scale, ln_bias, W1, b1, W2, b2):
    E = edge_type.shape[0]
    T = edge_ab.shape[0]
    D = fact_table.shape[1]
    L = W.shape[0]
    R2 = fact_table.shape[0]

    nblk = E // BLK
    ac_s, ab_s, bc_s = lax.sort(
        (edge_ac.astype(jnp.int32), edge_ab.astype(jnp.int32),
         edge_bc.astype(jnp.int32)), num_keys=1)
    acl_s = ac_s - (ac_s // BLK) * BLK
    bs = jnp.searchsorted(ac_s, jnp.arange(nblk + 1, dtype=jnp.int32) * BLK,
                          side="left").astype(jnp.int32)
    nbs = ((nblk + 1 + 15) // 16) * 16
    bs = jnp.concatenate([bs, jnp.full((nbs - (nblk + 1),), T, jnp.int32)])
    T_pad = T + C
    zpad = jnp.zeros((T_pad - T,), jnp.int32)
    ab_s = jnp.concatenate([ab_s, zpad])
    bc_s = jnp.concatenate([bc_s, zpad])
    acl_s = jnp.concatenate([acl_s, zpad])
    idx3 = jnp.stack([ab_s, bc_s, acl_s])

    msg_k = _message_kernel(E, D, T_pad, nblk, nbs)

    RB = 1000
    grid = (E // RB,)
    tc_layer = pl.pallas_call(
        _tc_layer_body,
        grid=grid,
        in_specs=[
            pl.BlockSpec((RB, D), lambda i: (i, 0)),
            pl.BlockSpec((RB, D), lambda i: (i, 0)),
            pl.BlockSpec((D, D), lambda i: (0, 0)),
            pl.BlockSpec((D,), lambda i: (0,)),
            pl.BlockSpec((D,), lambda i: (0,)),
            pl.BlockSpec((D,), lambda i: (0,)),
        ],
        out_specs=pl.BlockSpec((RB, D), lambda i: (i, 0)),
        out_shape=jax.ShapeDtypeStruct((E, D), jnp.float32),
    )

    KP = 128
    factp = jnp.zeros((KP, D), jnp.float32).at[:R2].set(fact_table)
    boundary = pl.pallas_call(
        _tc_boundary_body,
        grid=grid,
        in_specs=[
            pl.BlockSpec((RB, 1), lambda i: (i, 0)),
            pl.BlockSpec((KP, D), lambda i: (0, 0)),
        ],
        out_specs=pl.BlockSpec((RB, D), lambda i: (i, 0)),
        out_shape=jax.ShapeDtypeStruct((E, D), jnp.float32),
    )(edge_type.astype(jnp.int32).reshape(E, 1), factp)

    hidden = boundary
    for i in range(L - 1):
        aggb = msg_k(hidden, boundary, idx3, bs)
        hidden = tc_layer(aggb, hidden, W[i], b[i], ln_scale[i], ln_bias[i])

    aggb = msg_k(hidden, boundary, idx3, bs)
    query = query_table[r_index]
    w1a = W1[:D]
    w1b = W1[D:]
    i5 = L - 1
    score = pl.pallas_call(
        _tc_final_body,
        grid=grid,
        in_specs=[
            pl.BlockSpec((RB, D), lambda i: (i, 0)),
            pl.BlockSpec((RB, D), lambda i: (i, 0)),
            pl.BlockSpec((D, D), lambda i: (0, 0)),
            pl.BlockSpec((D,), lambda i: (0,)),
            pl.BlockSpec((D,), lambda i: (0,)),
            pl.BlockSpec((D,), lambda i: (0,)),
            pl.BlockSpec((D, 2 * D), lambda i: (0, 0)),
            pl.BlockSpec((D, 2 * D), lambda i: (0, 0)),
            pl.BlockSpec((2 * D,), lambda i: (0,)),
            pl.BlockSpec((1, D), lambda i: (0, 0)),
            pl.BlockSpec((2 * D, 1), lambda i: (0, 0)),
            pl.BlockSpec((1, 1), lambda i: (0, 0)),
        ],
        out_specs=pl.BlockSpec((RB, 1), lambda i: (i, 0)),
        out_shape=jax.ShapeDtypeStruct((E, 1), jnp.float32),
    )(aggb, hidden, W[i5], b[i5], ln_scale[i5], ln_bias[i5],
      w1a, w1b, b1, query, W2, b2.reshape(1, 1))
    return score


# speedup vs baseline: 2.0910x; 1.0211x over previous
import functools

import jax
import jax.numpy as jnp
from jax import lax
from jax.experimental import pallas as pl
from jax.experimental.pallas import tpu as pltpu
from jax.experimental.pallas import tpu_sc as plsc

NC = 2
NS = 16
NW = NC * NS
LANES = 16
BLK = 250
C = 32
CB = 128


def _sc_mesh():
    return plsc.VectorSubcoreMesh(
        core_axis_name="c", subcore_axis_name="s", num_cores=NC, num_subcores=NS
    )


def _tc_boundary_body(type_ref, factp_ref, out_ref):
    rb = type_ref.shape[0]
    kp = factp_ref.shape[0]
    oh = (type_ref[...] == lax.broadcasted_iota(jnp.int32, (rb, kp), 1))
    out_ref[...] = jnp.dot(oh.astype(jnp.float32), factp_ref[...],
                           preferred_element_type=jnp.float32)


def _message_kernel(E, D, T_pad, nblk, nbs):
    npass = (nblk + NW - 1) // NW

    STG = 125
    TRASH = NS * BLK

    def body(hidden_hbm, boundary_hbm, idx3_hbm, bs_hbm,
             out_hbm, agg_sh, ra0_v, rb0_v, ra1_v, rb1_v, stg_v,
             ix0_v, ix1_v, il0_v, il1_v, bs_v,
             sem_i0, sem_i1, sem_a0, sem_b0, sem_a1, sem_b1,
             sem_s0, sem_s1):
        cid = lax.axis_index("c")
        sid = lax.axis_index("s")
        wid = sid * NC + cid
        pltpu.sync_copy(bs_hbm, bs_v)
        iota = lax.broadcasted_iota(jnp.int32, (LANES,), 0)
        ras = (ra0_v, ra1_v)
        rbs = (rb0_v, rb1_v)
        ixs = (ix0_v, ix1_v)
        ils = (il0_v, il1_v)
        sis = (sem_i0, sem_i1)
        sas = (sem_a0, sem_a1)
        sbs = (sem_b0, sem_b1)
        sss = (sem_s0, sem_s1)

        def pass_body(p, carry):
            db = p * NW + wid

            @pl.when(db < nblk)
            def _():
                bsv = bs_v[pl.ds(db, LANES)]
                t0 = bsv[0]
                t1 = bsv[1]
                base_row = db * BLK
                arow = sid * BLK
                for j in range(BLK // STG):
                    pltpu.sync_copy(
                        boundary_hbm.at[pl.ds(base_row + j * STG, STG)], stg_v
                    )
                    pltpu.sync_copy(stg_v, agg_sh.at[pl.ds(arow + j * STG, STG)])
                t0a = (t0 // 8) * 8
                nch = (t1 - t0a + C - 1) // C
                t0v = jnp.broadcast_to(t0, (LANES,))
                t1v = jnp.broadcast_to(t1, (LANES,))

                def fire_idx(k, bb):
                    @pl.when(k < nch)
                    def _():
                        cb = t0a + k * C
                        pltpu.async_copy(
                            idx3_hbm.at[:, pl.ds(cb, C)], ixs[bb], sis[bb]
                        )

                def fire_rows(k, bb):
                    @pl.when(k < nch)
                    def _():
                        cb = t0a + k * C

                        @pl.when(k >= 2)
                        def _():
                            pltpu.make_async_copy(
                                ras[bb], agg_sh.at[ils[bb]], sss[bb]
                            ).wait()

                        pltpu.make_async_copy(
                            idx3_hbm.at[:, pl.ds(cb, C)], ixs[bb], sis[bb]
                        ).wait()
                        pltpu.async_copy(
                            hidden_hbm.at[ixs[bb].at[0]], ras[bb], sas[bb]
                        )
                        pltpu.async_copy(
                            hidden_hbm.at[ixs[bb].at[1]], rbs[bb], sbs[bb]
                        )
                        for q in range(C // LANES):
                            s = pl.ds(q * LANES, LANES)
                            tgv = iota + (cb + q * LANES)
                            ok = (tgv >= t0v) & (tgv < t1v)
                            ils[bb][s] = jnp.where(
                                ok, ixs[bb][2, s] + arow, TRASH
                            )

                def wait_rows(k, bb):
                    @pl.when(k < nch)
                    def _():
                        pltpu.make_async_copy(
                            hidden_hbm.at[ixs[bb].at[0]], ras[bb], sas[bb]
                        ).wait()
                        pltpu.make_async_copy(
                            hidden_hbm.at[ixs[bb].at[1]], rbs[bb], sbs[bb]
                        ).wait()

                def compute(k, bb):
                    @pl.when(k < nch)
                    def _():
                        def mrow(r4, carry3):
                            for r16 in range(4):
                                r = r4 * 4 + r16
                                for dd in range(D // LANES):
                                    s = pl.ds(dd * LANES, LANES)
                                    ras[bb][r, s] = ras[bb][r, s] * rbs[bb][r, s]
                            return carry3

                        lax.fori_loop(0, C // 4, mrow, 0)
                        pltpu.async_copy(
                            ras[bb], agg_sh.at[ils[bb]], sss[bb], add=True
                        )

                fire_idx(0, 0)
                fire_idx(1, 1)
                fire_rows(0, 0)

                def pair(g, carry2):
                    k = 2 * g
                    wait_rows(k, 0)
                    fire_idx(k + 2, 0)
                    fire_rows(k + 1, 1)
                    compute(k, 0)
                    wait_rows(k + 1, 1)
                    fire_idx(k + 3, 1)
                    fire_rows(k + 2, 0)
                    compute(k + 1, 1)
                    return carry2

                lax.fori_loop(0, (nch + 1) // 2, pair, 0)

                @pl.when(nch >= 1)
                def _():
                    pltpu.make_async_copy(
                        ras[0], agg_sh.at[ils[0]], sss[0]
                    ).wait()

                @pl.when(nch >= 2)
                def _():
                    pltpu.make_async_copy(
                        ras[1], agg_sh.at[ils[1]], sss[1]
                    ).wait()

                for j in range(BLK // STG):
                    pltpu.sync_copy(agg_sh.at[pl.ds(arow + j * STG, STG)], stg_v)
                    pltpu.sync_copy(
                        stg_v, out_hbm.at[pl.ds(base_row + j * STG, STG)]
                    )

            return carry

        lax.fori_loop(0, npass, pass_body, 0)

    return pl.kernel(
        body,
        out_type=jax.ShapeDtypeStruct((E, D), jnp.float32),
        mesh=_sc_mesh(),
        scratch_types=[
            pltpu.VMEM_SHARED((NS * BLK + 8, D), jnp.float32),
            pltpu.VMEM((C, D), jnp.float32),
            pltpu.VMEM((C, D), jnp.float32),
            pltpu.VMEM((C, D), jnp.float32),
            pltpu.VMEM((C, D), jnp.float32),
            pltpu.VMEM((STG, D), jnp.float32),
            pltpu.VMEM((3, C), jnp.int32),
            pltpu.VMEM((3, C), jnp.int32),
            pltpu.VMEM((C,), jnp.int32),
            pltpu.VMEM((C,), jnp.int32),
            pltpu.VMEM((nbs,), jnp.int32),
            pltpu.SemaphoreType.DMA,
            pltpu.SemaphoreType.DMA,
            pltpu.SemaphoreType.DMA,
            pltpu.SemaphoreType.DMA,
            pltpu.SemaphoreType.DMA,
            pltpu.SemaphoreType.DMA,
            pltpu.SemaphoreType.DMA,
            pltpu.SemaphoreType.DMA,
        ],
        compiler_params=pltpu.CompilerParams(use_tc_tiling_on_sc=False),
    )


def _tc_layer_body(aggb_ref, hid_ref, w_ref, b_ref, s_ref, t_ref, out_ref):
    y = jnp.dot(aggb_ref[...], w_ref[...], preferred_element_type=jnp.float32)
    y = y + b_ref[...]
    mean = jnp.mean(y, axis=1, keepdims=True)
    var = jnp.mean((y - mean) ** 2, axis=1, keepdims=True)
    yn = (y - mean) * lax.rsqrt(var + 1e-5) * s_ref[...] + t_ref[...]
    out_ref[...] = jnp.maximum(yn, 0.0) + hid_ref[...]


def _tc_final_body(aggb_ref, hid_ref, w_ref, b_ref, s_ref, t_ref,
                   w1a_ref, w1b_ref, b1_ref, q_ref, w2_ref, b2_ref, out_ref):
    y = jnp.dot(aggb_ref[...], w_ref[...], preferred_element_type=jnp.float32)
    y = y + b_ref[...]
    mean = jnp.mean(y, axis=1, keepdims=True)
    var = jnp.mean((y - mean) ** 2, axis=1, keepdims=True)
    yn = (y - mean) * lax.rsqrt(var + 1e-5) * s_ref[...] + t_ref[...]
    hid = jnp.maximum(yn, 0.0) + hid_ref[...]
    qv = jnp.dot(q_ref[...], w1b_ref[...], preferred_element_type=jnp.float32)
    qv = qv + b1_ref[...]
    h1 = jnp.dot(hid, w1a_ref[...], preferred_element_type=jnp.float32)
    h1 = jnp.maximum(h1 + qv, 0.0)
    out_ref[...] = (
        jnp.dot(h1, w2_ref[...], preferred_element_type=jnp.float32) + b2_ref[...]
    )


def kernel(edge_type, edge_ab, edge_bc, edge_ac, r_index, query_table,
           fact_table, W, b, ln_scale, ln_bias, W1, b1, W2, b2):
    E = edge_type.shape[0]
    T = edge_ab.shape[0]
    D = fact_table.shape[1]
    L = W.shape[0]
    R2 = fact_table.shape[0]

    nblk = E // BLK
    ac_s, ab_s, bc_s = lax.sort(
        (edge_ac.astype(jnp.int32), edge_ab.astype(jnp.int32),
         edge_bc.astype(jnp.int32)), num_keys=1)
    acl_s = ac_s - (ac_s // BLK) * BLK
    bs = jnp.searchsorted(ac_s, jnp.arange(nblk + 1, dtype=jnp.int32) * BLK,
                          side="left").astype(jnp.int32)
    nbs = ((nblk + 1 + 15) // 16) * 16
    bs = jnp.concatenate([bs, jnp.full((nbs - (nblk + 1),), T, jnp.int32)])
    T_pad = T + C
    zpad = jnp.zeros((T_pad - T,), jnp.int32)
    ab_s = jnp.concatenate([ab_s, zpad])
    bc_s = jnp.concatenate([bc_s, zpad])
    acl_s = jnp.concatenate([acl_s, zpad])
    idx3 = jnp.stack([ab_s, bc_s, acl_s])

    msg_k = _message_kernel(E, D, T_pad, nblk, nbs)

    RB = 1000
    grid = (E // RB,)
    tc_layer = pl.pallas_call(
        _tc_layer_body,
        grid=grid,
        in_specs=[
            pl.BlockSpec((RB, D), lambda i: (i, 0)),
            pl.BlockSpec((RB, D), lambda i: (i, 0)),
            pl.BlockSpec((D, D), lambda i: (0, 0)),
            pl.BlockSpec((D,), lambda i: (0,)),
            pl.BlockSpec((D,), lambda i: (0,)),
            pl.BlockSpec((D,), lambda i: (0,)),
        ],
        out_specs=pl.BlockSpec((RB, D), lambda i: (i, 0)),
        out_shape=jax.ShapeDtypeStruct((E, D), jnp.float32),
    )

    KP = 128
    factp = jnp.zeros((KP, D), jnp.float32).at[:R2].set(fact_table)
    boundary = pl.pallas_call(
        _tc_boundary_body,
        grid=grid,
        in_specs=[
            pl.BlockSpec((RB, 1), lambda i: (i, 0)),
            pl.BlockSpec((KP, D), lambda i: (0, 0)),
        ],
        out_specs=pl.BlockSpec((RB, D), lambda i: (i, 0)),
        out_shape=jax.ShapeDtypeStruct((E, D), jnp.float32),
    )(edge_type.astype(jnp.int32).reshape(E, 1), factp)

    hidden = boundary
    for i in range(L - 1):
        aggb = msg_k(hidden, boundary, idx3, bs)
        hidden = tc_layer(aggb, hidden, W[i], b[i], ln_scale[i], ln_bias[i])

    aggb = msg_k(hidden, boundary, idx3, bs)
    query = query_table[r_index]
    w1a = W1[:D]
    w1b = W1[D:]
    i5 = L - 1
    score = pl.pallas_call(
        _tc_final_body,
        grid=grid,
        in_specs=[
            pl.BlockSpec((RB, D), lambda i: (i, 0)),
            pl.BlockSpec((RB, D), lambda i: (i, 0)),
            pl.BlockSpec((D, D), lambda i: (0, 0)),
            pl.BlockSpec((D,), lambda i: (0,)),
            pl.BlockSpec((D,), lambda i: (0,)),
            pl.BlockSpec((D,), lambda i: (0,)),
            pl.BlockSpec((D, 2 * D), lambda i: (0, 0)),
            pl.BlockSpec((D, 2 * D), lambda i: (0, 0)),
            pl.BlockSpec((2 * D,), lambda i: (0,)),
            pl.BlockSpec((1, D), lambda i: (0, 0)),
            pl.BlockSpec((2 * D, 1), lambda i: (0, 0)),
            pl.BlockSpec((1, 1), lambda i: (0, 0)),
        ],
        out_specs=pl.BlockSpec((RB, 1), lambda i: (i, 0)),
        out_shape=jax.ShapeDtypeStruct((E, 1), jnp.float32),
    )(aggb, hidden, W[i5], b[i5], ln_scale[i5], ln_bias[i5],
      w1a, w1b, b1, query, W2, b2.reshape(1, 1))
    return score
